# Initial kernel scaffold; baseline (speedup 1.0000x reference)
import jax
import jax.numpy as jnp
from jax.experimental import pallas as pl


def kernel(x, edge_index, W1, b1, W2, b2, Wp1, bp1, Wp2, bp2):
    raise NotImplementedError("write your pallas kernel here")


# speedup vs baseline: 7.8868x; 7.8868x over previous
import functools

import jax
import jax.numpy as jnp
from jax import lax
from jax.experimental import pallas as pl
from jax.experimental.pallas import tpu as pltpu
from jax.experimental.pallas import tpu_sc as plsc

N = 10000
E = 320000
D = 128

NC = 2
NS = 16
NW = NC * NS
EPW = E // NW
K = 100
CH = EPW // K
RPT = 624
REM = N - NS * RPT

BM = 1000


def _lin_body(x_ref, w_ref, b_ref, o_ref):
    o_ref[...] = (
        jnp.dot(x_ref[...], w_ref[...], preferred_element_type=jnp.float32)
        + b_ref[...]
    )


def _linear(x, W, b):
    return pl.pallas_call(
        _lin_body,
        grid=(N // BM,),
        in_specs=[
            pl.BlockSpec((BM, D), lambda i: (i, 0)),
            pl.BlockSpec((D, D), lambda i: (0, 0)),
            pl.BlockSpec((1, D), lambda i: (0, 0)),
        ],
        out_specs=pl.BlockSpec((BM, D), lambda i: (i, 0)),
        out_shape=jax.ShapeDtypeStruct((N, D), jnp.float32),
    )(x, W, b.reshape(1, D))


def _relu_sum_lin_body(p_ref, w_ref, b_ref, o_ref):
    h = jnp.maximum(p_ref[0] + p_ref[1], 0.0)
    o_ref[...] = (
        jnp.dot(h, w_ref[...], preferred_element_type=jnp.float32) + b_ref[...]
    )


def _relu_sum_linear(p, W, b):
    return pl.pallas_call(
        _relu_sum_lin_body,
        grid=(N // BM,),
        in_specs=[
            pl.BlockSpec((NC, BM, D), lambda i: (0, i, 0)),
            pl.BlockSpec((D, D), lambda i: (0, 0)),
            pl.BlockSpec((1, D), lambda i: (0, 0)),
        ],
        out_specs=pl.BlockSpec((BM, D), lambda i: (i, 0)),
        out_shape=jax.ShapeDtypeStruct((N, D), jnp.float32),
    )(p, W, b.reshape(1, D))


def _final_body(p_ref, w1_ref, b1_ref, w2_ref, b2_ref, o_ref):
    h = jnp.maximum(p_ref[0] + p_ref[1], 0.0)
    t = jnp.dot(h, w1_ref[...], preferred_element_type=jnp.float32) + b1_ref[...]
    o = jnp.dot(t, w2_ref[...], preferred_element_type=jnp.float32) + b2_ref[...]
    m = jnp.max(o, axis=1, keepdims=True)
    lse = jnp.log(jnp.sum(jnp.exp(o - m), axis=1, keepdims=True)) + m
    o_ref[...] = o - lse


def _final(p, Wp1, bp1, Wp2, bp2):
    return pl.pallas_call(
        _final_body,
        grid=(N // BM,),
        in_specs=[
            pl.BlockSpec((NC, BM, D), lambda i: (0, i, 0)),
            pl.BlockSpec((D, D), lambda i: (0, 0)),
            pl.BlockSpec((1, D), lambda i: (0, 0)),
            pl.BlockSpec((D, D), lambda i: (0, 0)),
            pl.BlockSpec((1, D), lambda i: (0, 0)),
        ],
        out_specs=pl.BlockSpec((BM, D), lambda i: (i, 0)),
        out_shape=jax.ShapeDtypeStruct((N, D), jnp.float32),
    )(p, Wp1, bp1.reshape(1, D), Wp2, bp2.reshape(1, D))


def _spmm_partials(hid, col_g, row_g):
    mesh = plsc.VectorSubcoreMesh(
        core_axis_name="c", subcore_axis_name="s",
        num_cores=NC, num_subcores=NS,
    )

    @functools.partial(
        pl.kernel,
        out_type=jax.ShapeDtypeStruct((NC, N, D), jnp.float32),
        mesh=mesh,
        scratch_types=[
            pltpu.VMEM((CH, K), jnp.int32),
            pltpu.VMEM((CH, K), jnp.int32),
            pltpu.VMEM((K, D), jnp.float32),
            pltpu.VMEM_SHARED((N, D), jnp.float32),
            pltpu.SemaphoreType.DMA,
        ],
    )
    def k(hid_hbm, col_hbm, row_hbm, out_hbm,
          col_v, row_v, rows_v, acc_sh, sem):
        cid = lax.axis_index("c")
        sid = lax.axis_index("s")
        wid = sid * NC + cid

        z = jnp.zeros((16,), jnp.float32)

        def zb(i, carry):
            for cc in range(D // 16):
                rows_v[i, pl.ds(cc * 16, 16)] = z
            return carry

        lax.fori_loop(0, K, zb, 0)
        for r in range(RPT // 96):
            pltpu.sync_copy(rows_v.at[pl.ds(0, 96)],
                            acc_sh.at[pl.ds(sid * RPT + r * 96, 96)])
        pltpu.sync_copy(rows_v.at[pl.ds(0, RPT - 6 * 96)],
                        acc_sh.at[pl.ds(sid * RPT + 6 * 96, RPT - 6 * 96)])

        @pl.when(sid == NS - 1)
        def _zero_tail():
            pltpu.sync_copy(rows_v.at[pl.ds(0, REM)],
                            acc_sh.at[pl.ds(NS * RPT, REM)])

        plsc.subcore_barrier()

        pltpu.sync_copy(col_hbm.at[wid], col_v)
        pltpu.sync_copy(row_hbm.at[wid], row_v)

        def body(j, carry):
            pltpu.async_copy(hid_hbm.at[col_v.at[j]], rows_v, sem).wait()
            pltpu.sync_copy(rows_v, acc_sh.at[row_v.at[j]], add=True)
            return carry

        lax.fori_loop(0, CH, body, 0)
        plsc.subcore_barrier()

        pltpu.sync_copy(acc_sh.at[pl.ds(sid * RPT, RPT)],
                        out_hbm.at[cid, pl.ds(sid * RPT, RPT)])

        @pl.when(sid == NS - 1)
        def _wb_tail():
            pltpu.sync_copy(acc_sh.at[pl.ds(NS * RPT, REM)],
                            out_hbm.at[cid, pl.ds(NS * RPT, REM)])

    return k(hid, col_g, row_g)


def kernel(x, edge_index, W1, b1, W2, b2, Wp1, bp1, Wp2, bp2):
    col_g = edge_index[1].reshape(NW, CH, K)
    row_g = edge_index[0].reshape(NW, CH, K)

    hid1 = _linear(x, W1, b1)
    p1 = _spmm_partials(hid1, col_g, row_g)
    hid2 = _relu_sum_linear(p1, W2, b2)
    p2 = _spmm_partials(hid2, col_g, row_g)
    return _final(p2, Wp1, bp1, Wp2, bp2)


# speedup vs baseline: 11.9155x; 1.5108x over previous
import functools

import jax
import jax.numpy as jnp
from jax import lax
from jax.experimental import pallas as pl
from jax.experimental.pallas import tpu as pltpu
from jax.experimental.pallas import tpu_sc as plsc

N = 10000
E = 320000
D = 128

NC = 2
NS = 16
NW = NC * NS
EPW = E // NW
K = 100
CH = EPW // K
PH = 2
CHH = CH // PH
RPT = 624
REM = N - NS * RPT

BM = 1000


def _lin_body(x_ref, w_ref, b_ref, o_ref):
    o_ref[...] = (
        jnp.dot(x_ref[...], w_ref[...], preferred_element_type=jnp.float32)
        + b_ref[...]
    )


def _linear(x, W, b):
    return pl.pallas_call(
        _lin_body,
        grid=(N // BM,),
        in_specs=[
            pl.BlockSpec((BM, D), lambda i: (i, 0)),
            pl.BlockSpec((D, D), lambda i: (0, 0)),
            pl.BlockSpec((1, D), lambda i: (0, 0)),
        ],
        out_specs=pl.BlockSpec((BM, D), lambda i: (i, 0)),
        out_shape=jax.ShapeDtypeStruct((N, D), jnp.float32),
    )(x, W, b.reshape(1, D))


def _relu_sum_lin_body(p_ref, w_ref, b_ref, o_ref):
    h = jnp.maximum(p_ref[0] + p_ref[1], 0.0)
    o_ref[...] = (
        jnp.dot(h, w_ref[...], preferred_element_type=jnp.float32) + b_ref[...]
    )


def _relu_sum_linear(p, W, b):
    return pl.pallas_call(
        _relu_sum_lin_body,
        grid=(N // BM,),
        in_specs=[
            pl.BlockSpec((NC, BM, D), lambda i: (0, i, 0)),
            pl.BlockSpec((D, D), lambda i: (0, 0)),
            pl.BlockSpec((1, D), lambda i: (0, 0)),
        ],
        out_specs=pl.BlockSpec((BM, D), lambda i: (i, 0)),
        out_shape=jax.ShapeDtypeStruct((N, D), jnp.float32),
    )(p, W, b.reshape(1, D))


def _final_body(p_ref, w1_ref, b1_ref, w2_ref, b2_ref, o_ref):
    h = jnp.maximum(p_ref[0] + p_ref[1], 0.0)
    t = jnp.dot(h, w1_ref[...], preferred_element_type=jnp.float32) + b1_ref[...]
    o = jnp.dot(t, w2_ref[...], preferred_element_type=jnp.float32) + b2_ref[...]
    m = jnp.max(o, axis=1, keepdims=True)
    lse = jnp.log(jnp.sum(jnp.exp(o - m), axis=1, keepdims=True)) + m
    o_ref[...] = o - lse


def _final(p, Wp1, bp1, Wp2, bp2):
    return pl.pallas_call(
        _final_body,
        grid=(N // BM,),
        in_specs=[
            pl.BlockSpec((NC, BM, D), lambda i: (0, i, 0)),
            pl.BlockSpec((D, D), lambda i: (0, 0)),
            pl.BlockSpec((1, D), lambda i: (0, 0)),
            pl.BlockSpec((D, D), lambda i: (0, 0)),
            pl.BlockSpec((1, D), lambda i: (0, 0)),
        ],
        out_specs=pl.BlockSpec((BM, D), lambda i: (i, 0)),
        out_shape=jax.ShapeDtypeStruct((N, D), jnp.float32),
    )(p, Wp1, bp1.reshape(1, D), Wp2, bp2.reshape(1, D))


def _spmm_partials(hid, col_g, row_g):
    mesh = plsc.VectorSubcoreMesh(
        core_axis_name="c", subcore_axis_name="s",
        num_cores=NC, num_subcores=NS,
    )

    @functools.partial(
        pl.kernel,
        out_type=jax.ShapeDtypeStruct((NC, N, D), jnp.float32),
        mesh=mesh,
        scratch_types=[
            pltpu.VMEM((CHH, K), jnp.int32),
            pltpu.VMEM((CHH, K), jnp.int32),
            pltpu.VMEM((K, D), jnp.float32),
            pltpu.VMEM((K, D), jnp.float32),
            pltpu.VMEM_SHARED((N, D), jnp.float32),
            pltpu.SemaphoreType.DMA,
            pltpu.SemaphoreType.DMA,
            pltpu.SemaphoreType.DMA,
            pltpu.SemaphoreType.DMA,
        ],
    )
    def k(hid_hbm, col_hbm, row_hbm, out_hbm,
          col_v, row_v, rows_v0, rows_v1, acc_sh,
          gsem0, gsem1, ssem0, ssem1):
        rows_v = rows_v0
        bufs = (rows_v0, rows_v1)
        gsems = (gsem0, gsem1)
        ssems = (ssem0, ssem1)
        cid = lax.axis_index("c")
        sid = lax.axis_index("s")
        wid = sid * NC + cid

        z = jnp.zeros((16,), jnp.float32)

        def zb(i, carry):
            for cc in range(D // 16):
                rows_v[i, pl.ds(cc * 16, 16)] = z
            return carry

        lax.fori_loop(0, K, zb, 0)
        for r in range(RPT // 48):
            pltpu.sync_copy(rows_v.at[pl.ds(0, 48)],
                            acc_sh.at[pl.ds(sid * RPT + r * 48, 48)])

        @pl.when(sid == NS - 1)
        def _zero_tail():
            pltpu.sync_copy(rows_v.at[pl.ds(0, REM)],
                            acc_sh.at[pl.ds(NS * RPT, REM)])

        plsc.subcore_barrier()

        for h in range(PH):
            pltpu.sync_copy(col_hbm.at[wid, h], col_v)
            pltpu.sync_copy(row_hbm.at[wid, h], row_v)

            pltpu.async_copy(hid_hbm.at[col_v.at[0]], bufs[0], gsems[0])
            pltpu.async_copy(hid_hbm.at[col_v.at[1]], bufs[1], gsems[1])

            def body(jj, carry):
                for b in range(2):
                    j = 2 * jj + b
                    pltpu.make_async_copy(
                        hid_hbm.at[col_v.at[j]], bufs[b], gsems[b]).wait()
                    pltpu.async_copy(
                        bufs[b], acc_sh.at[row_v.at[j]], ssems[b], add=True
                    ).wait()
                    pltpu.async_copy(
                        hid_hbm.at[col_v.at[j + 2]], bufs[b], gsems[b])
                return carry

            lax.fori_loop(0, CHH // 2 - 1, body, 0)
            for b in range(2):
                j = CHH - 2 + b
                pltpu.make_async_copy(
                    hid_hbm.at[col_v.at[j]], bufs[b], gsems[b]).wait()
                pltpu.async_copy(
                    bufs[b], acc_sh.at[row_v.at[j]], ssems[b], add=True
                ).wait()
        plsc.subcore_barrier()

        pltpu.sync_copy(acc_sh.at[pl.ds(sid * RPT, RPT)],
                        out_hbm.at[cid, pl.ds(sid * RPT, RPT)])

        @pl.when(sid == NS - 1)
        def _wb_tail():
            pltpu.sync_copy(acc_sh.at[pl.ds(NS * RPT, REM)],
                            out_hbm.at[cid, pl.ds(NS * RPT, REM)])

    return k(hid, col_g, row_g)


def kernel(x, edge_index, W1, b1, W2, b2, Wp1, bp1, Wp2, bp2):
    col_g = edge_index[1].reshape(NW, PH, CHH, K)
    row_g = edge_index[0].reshape(NW, PH, CHH, K)

    hid1 = _linear(x, W1, b1)
    p1 = _spmm_partials(hid1, col_g, row_g)
    hid2 = _relu_sum_linear(p1, W2, b2)
    p2 = _spmm_partials(hid2, col_g, row_g)
    return _final(p2, Wp1, bp1, Wp2, bp2)


# speedup vs baseline: 12.1987x; 1.0238x over previous
import functools

import jax
import jax.numpy as jnp
from jax import lax
from jax.experimental import pallas as pl
from jax.experimental.pallas import tpu as pltpu
from jax.experimental.pallas import tpu_sc as plsc

N = 10000
E = 320000
D = 128

NC = 2
NS = 16
NW = NC * NS
EPW = E // NW
K = 125
CH = EPW // K
PH = 2
CHH = CH // PH
RPT = 624
REM = N - NS * RPT

BM = 1000


def _lin_body(x_ref, w_ref, b_ref, o_ref):
    o_ref[...] = (
        jnp.dot(x_ref[...], w_ref[...], preferred_element_type=jnp.float32)
        + b_ref[...]
    )


def _linear(x, W, b):
    return pl.pallas_call(
        _lin_body,
        grid=(N // BM,),
        in_specs=[
            pl.BlockSpec((BM, D), lambda i: (i, 0)),
            pl.BlockSpec((D, D), lambda i: (0, 0)),
            pl.BlockSpec((1, D), lambda i: (0, 0)),
        ],
        out_specs=pl.BlockSpec((BM, D), lambda i: (i, 0)),
        out_shape=jax.ShapeDtypeStruct((N, D), jnp.float32),
    )(x, W, b.reshape(1, D))


def _relu_sum_lin_body(p_ref, w_ref, b_ref, o_ref):
    h = jnp.maximum(p_ref[0] + p_ref[1], 0.0)
    o_ref[...] = (
        jnp.dot(h, w_ref[...], preferred_element_type=jnp.float32) + b_ref[...]
    )


def _relu_sum_linear(p, W, b):
    return pl.pallas_call(
        _relu_sum_lin_body,
        grid=(N // BM,),
        in_specs=[
            pl.BlockSpec((NC, BM, D), lambda i: (0, i, 0)),
            pl.BlockSpec((D, D), lambda i: (0, 0)),
            pl.BlockSpec((1, D), lambda i: (0, 0)),
        ],
        out_specs=pl.BlockSpec((BM, D), lambda i: (i, 0)),
        out_shape=jax.ShapeDtypeStruct((N, D), jnp.float32),
    )(p, W, b.reshape(1, D))


def _final_body(p_ref, w1_ref, b1_ref, w2_ref, b2_ref, o_ref):
    h = jnp.maximum(p_ref[0] + p_ref[1], 0.0)
    t = jnp.dot(h, w1_ref[...], preferred_element_type=jnp.float32) + b1_ref[...]
    o = jnp.dot(t, w2_ref[...], preferred_element_type=jnp.float32) + b2_ref[...]
    m = jnp.max(o, axis=1, keepdims=True)
    lse = jnp.log(jnp.sum(jnp.exp(o - m), axis=1, keepdims=True)) + m
    o_ref[...] = o - lse


def _final(p, Wp1, bp1, Wp2, bp2):
    return pl.pallas_call(
        _final_body,
        grid=(N // BM,),
        in_specs=[
            pl.BlockSpec((NC, BM, D), lambda i: (0, i, 0)),
            pl.BlockSpec((D, D), lambda i: (0, 0)),
            pl.BlockSpec((1, D), lambda i: (0, 0)),
            pl.BlockSpec((D, D), lambda i: (0, 0)),
            pl.BlockSpec((1, D), lambda i: (0, 0)),
        ],
        out_specs=pl.BlockSpec((BM, D), lambda i: (i, 0)),
        out_shape=jax.ShapeDtypeStruct((N, D), jnp.float32),
    )(p, Wp1, bp1.reshape(1, D), Wp2, bp2.reshape(1, D))


def _spmm_partials(hid, col_g, row_g):
    mesh = plsc.VectorSubcoreMesh(
        core_axis_name="c", subcore_axis_name="s",
        num_cores=NC, num_subcores=NS,
    )

    @functools.partial(
        pl.kernel,
        out_type=jax.ShapeDtypeStruct((NC, N, D), jnp.float32),
        mesh=mesh,
        scratch_types=[
            pltpu.VMEM((CHH, K), jnp.int32),
            pltpu.VMEM((CHH, K), jnp.int32),
            pltpu.VMEM((K, D), jnp.float32),
            pltpu.VMEM((K, D), jnp.float32),
            pltpu.VMEM_SHARED((N, D), jnp.float32),
            pltpu.SemaphoreType.DMA,
            pltpu.SemaphoreType.DMA,
            pltpu.SemaphoreType.DMA,
            pltpu.SemaphoreType.DMA,
        ],
    )
    def k(hid_hbm, col_hbm, row_hbm, out_hbm,
          col_v, row_v, rows_v0, rows_v1, acc_sh,
          gsem0, gsem1, ssem0, ssem1):
        rows_v = rows_v0
        bufs = (rows_v0, rows_v1)
        gsems = (gsem0, gsem1)
        ssems = (ssem0, ssem1)
        cid = lax.axis_index("c")
        sid = lax.axis_index("s")
        wid = sid * NC + cid

        z = jnp.zeros((16,), jnp.float32)

        def zb(i, carry):
            for cc in range(D // 16):
                rows_v[i, pl.ds(cc * 16, 16)] = z
            return carry

        lax.fori_loop(0, K, zb, 0)
        for r in range(RPT // 48):
            pltpu.sync_copy(rows_v.at[pl.ds(0, 48)],
                            acc_sh.at[pl.ds(sid * RPT + r * 48, 48)])

        @pl.when(sid == NS - 1)
        def _zero_tail():
            pltpu.sync_copy(rows_v.at[pl.ds(0, REM)],
                            acc_sh.at[pl.ds(NS * RPT, REM)])

        plsc.subcore_barrier()

        for h in range(PH):
            pltpu.sync_copy(col_hbm.at[wid, h], col_v)
            pltpu.sync_copy(row_hbm.at[wid, h], row_v)

            pltpu.async_copy(hid_hbm.at[col_v.at[0]], bufs[0], gsems[0])
            pltpu.async_copy(hid_hbm.at[col_v.at[1]], bufs[1], gsems[1])

            def body(jj, carry):
                for b in range(2):
                    j = 2 * jj + b
                    pltpu.make_async_copy(
                        hid_hbm.at[col_v.at[j]], bufs[b], gsems[b]).wait()
                    pltpu.async_copy(
                        bufs[b], acc_sh.at[row_v.at[j]], ssems[b], add=True
                    ).wait()
                    pltpu.async_copy(
                        hid_hbm.at[col_v.at[j + 2]], bufs[b], gsems[b])
                return carry

            lax.fori_loop(0, CHH // 2 - 1, body, 0)
            for b in range(2):
                j = CHH - 2 + b
                pltpu.make_async_copy(
                    hid_hbm.at[col_v.at[j]], bufs[b], gsems[b]).wait()
                pltpu.async_copy(
                    bufs[b], acc_sh.at[row_v.at[j]], ssems[b], add=True
                ).wait()
        plsc.subcore_barrier()

        pltpu.sync_copy(acc_sh.at[pl.ds(sid * RPT, RPT)],
                        out_hbm.at[cid, pl.ds(sid * RPT, RPT)])

        @pl.when(sid == NS - 1)
        def _wb_tail():
            pltpu.sync_copy(acc_sh.at[pl.ds(NS * RPT, REM)],
                            out_hbm.at[cid, pl.ds(NS * RPT, REM)])

    return k(hid, col_g, row_g)


def kernel(x, edge_index, W1, b1, W2, b2, Wp1, bp1, Wp2, bp2):
    col_g = edge_index[1].reshape(NW, PH, CHH, K)
    row_g = edge_index[0].reshape(NW, PH, CHH, K)

    hid1 = _linear(x, W1, b1)
    p1 = _spmm_partials(hid1, col_g, row_g)
    hid2 = _relu_sum_linear(p1, W2, b2)
    p2 = _spmm_partials(hid2, col_g, row_g)
    return _final(p2, Wp1, bp1, Wp2, bp2)


# speedup vs baseline: 12.7343x; 1.0439x over previous
import functools

import jax
import jax.numpy as jnp
from jax import lax
from jax.experimental import pallas as pl
from jax.experimental.pallas import tpu as pltpu
from jax.experimental.pallas import tpu_sc as plsc

N = 10000
E = 320000
D = 128

NC = 2
NS = 16
NW = NC * NS
EPW = E // NW
K = 100
CH = EPW // K
PH = 4
CHH = CH // PH
NBUF = 3
RPT = 624
REM = N - NS * RPT

BM = 1000


def _lin_body(x_ref, w_ref, b_ref, o_ref):
    o_ref[...] = (
        jnp.dot(x_ref[...], w_ref[...], preferred_element_type=jnp.float32)
        + b_ref[...]
    )


def _linear(x, W, b):
    return pl.pallas_call(
        _lin_body,
        grid=(N // BM,),
        in_specs=[
            pl.BlockSpec((BM, D), lambda i: (i, 0)),
            pl.BlockSpec((D, D), lambda i: (0, 0)),
            pl.BlockSpec((1, D), lambda i: (0, 0)),
        ],
        out_specs=pl.BlockSpec((BM, D), lambda i: (i, 0)),
        out_shape=jax.ShapeDtypeStruct((N, D), jnp.float32),
    )(x, W, b.reshape(1, D))


def _relu_sum_lin_body(p_ref, w_ref, b_ref, o_ref):
    h = jnp.maximum(p_ref[0] + p_ref[1], 0.0)
    o_ref[...] = (
        jnp.dot(h, w_ref[...], preferred_element_type=jnp.float32) + b_ref[...]
    )


def _relu_sum_linear(p, W, b):
    return pl.pallas_call(
        _relu_sum_lin_body,
        grid=(N // BM,),
        in_specs=[
            pl.BlockSpec((NC, BM, D), lambda i: (0, i, 0)),
            pl.BlockSpec((D, D), lambda i: (0, 0)),
            pl.BlockSpec((1, D), lambda i: (0, 0)),
        ],
        out_specs=pl.BlockSpec((BM, D), lambda i: (i, 0)),
        out_shape=jax.ShapeDtypeStruct((N, D), jnp.float32),
    )(p, W, b.reshape(1, D))


def _final_body(p_ref, w1_ref, b1_ref, w2_ref, b2_ref, o_ref):
    h = jnp.maximum(p_ref[0] + p_ref[1], 0.0)
    t = jnp.dot(h, w1_ref[...], preferred_element_type=jnp.float32) + b1_ref[...]
    o = jnp.dot(t, w2_ref[...], preferred_element_type=jnp.float32) + b2_ref[...]
    m = jnp.max(o, axis=1, keepdims=True)
    lse = jnp.log(jnp.sum(jnp.exp(o - m), axis=1, keepdims=True)) + m
    o_ref[...] = o - lse


def _final(p, Wp1, bp1, Wp2, bp2):
    return pl.pallas_call(
        _final_body,
        grid=(N // BM,),
        in_specs=[
            pl.BlockSpec((NC, BM, D), lambda i: (0, i, 0)),
            pl.BlockSpec((D, D), lambda i: (0, 0)),
            pl.BlockSpec((1, D), lambda i: (0, 0)),
            pl.BlockSpec((D, D), lambda i: (0, 0)),
            pl.BlockSpec((1, D), lambda i: (0, 0)),
        ],
        out_specs=pl.BlockSpec((BM, D), lambda i: (i, 0)),
        out_shape=jax.ShapeDtypeStruct((N, D), jnp.float32),
    )(p, Wp1, bp1.reshape(1, D), Wp2, bp2.reshape(1, D))


def _spmm_partials(hid, col_g, row_g):
    mesh = plsc.VectorSubcoreMesh(
        core_axis_name="c", subcore_axis_name="s",
        num_cores=NC, num_subcores=NS,
    )

    @functools.partial(
        pl.kernel,
        out_type=jax.ShapeDtypeStruct((NC, N, D), jnp.float32),
        mesh=mesh,
        scratch_types=[
            pltpu.VMEM((CHH, K), jnp.int32),
            pltpu.VMEM((CHH, K), jnp.int32),
            pltpu.VMEM((K, D), jnp.float32),
            pltpu.VMEM((K, D), jnp.float32),
            pltpu.VMEM((K, D), jnp.float32),
            pltpu.VMEM_SHARED((N, D), jnp.float32),
            pltpu.SemaphoreType.DMA,
            pltpu.SemaphoreType.DMA,
            pltpu.SemaphoreType.DMA,
            pltpu.SemaphoreType.DMA,
            pltpu.SemaphoreType.DMA,
            pltpu.SemaphoreType.DMA,
        ],
    )
    def k(hid_hbm, col_hbm, row_hbm, out_hbm,
          col_v, row_v, rows_v0, rows_v1, rows_v2, acc_sh,
          gsem0, gsem1, gsem2, ssem0, ssem1, ssem2):
        rows_v = rows_v0
        bufs = (rows_v0, rows_v1, rows_v2)
        gsems = (gsem0, gsem1, gsem2)
        ssems = (ssem0, ssem1, ssem2)
        cid = lax.axis_index("c")
        sid = lax.axis_index("s")
        wid = sid * NC + cid

        z = jnp.zeros((16,), jnp.float32)

        def zb(i, carry):
            for cc in range(D // 16):
                rows_v[i, pl.ds(cc * 16, 16)] = z
            return carry

        lax.fori_loop(0, K, zb, 0)
        for r in range(RPT // 48):
            pltpu.sync_copy(rows_v.at[pl.ds(0, 48)],
                            acc_sh.at[pl.ds(sid * RPT + r * 48, 48)])

        @pl.when(sid == NS - 1)
        def _zero_tail():
            pltpu.sync_copy(rows_v.at[pl.ds(0, REM)],
                            acc_sh.at[pl.ds(NS * RPT, REM)])

        plsc.subcore_barrier()

        for h in range(PH):
            pltpu.sync_copy(col_hbm.at[wid, h], col_v)
            pltpu.sync_copy(row_hbm.at[wid, h], row_v)

            for j in range(NBUF):
                pltpu.async_copy(
                    hid_hbm.at[col_v.at[j]], bufs[j], gsems[j])

            for j in range(CHH):
                b = j % NBUF
                pltpu.make_async_copy(
                    hid_hbm.at[col_v.at[j]], bufs[b], gsems[b]).wait()
                pltpu.async_copy(
                    bufs[b], acc_sh.at[row_v.at[j]], ssems[b], add=True
                ).wait()
                if j + NBUF < CHH:
                    pltpu.async_copy(
                        hid_hbm.at[col_v.at[j + NBUF]], bufs[b], gsems[b])
        plsc.subcore_barrier()

        pltpu.sync_copy(acc_sh.at[pl.ds(sid * RPT, RPT)],
                        out_hbm.at[cid, pl.ds(sid * RPT, RPT)])

        @pl.when(sid == NS - 1)
        def _wb_tail():
            pltpu.sync_copy(acc_sh.at[pl.ds(NS * RPT, REM)],
                            out_hbm.at[cid, pl.ds(NS * RPT, REM)])

    return k(hid, col_g, row_g)


def kernel(x, edge_index, W1, b1, W2, b2, Wp1, bp1, Wp2, bp2):
    col_g = edge_index[1].reshape(NW, PH, CHH, K)
    row_g = edge_index[0].reshape(NW, PH, CHH, K)

    hid1 = _linear(x, W1, b1)
    p1 = _spmm_partials(hid1, col_g, row_g)
    hid2 = _relu_sum_linear(p1, W2, b2)
    p2 = _spmm_partials(hid2, col_g, row_g)
    return _final(p2, Wp1, bp1, Wp2, bp2)
